# baseline (speedup 1.0000x reference)
import functools

import jax
import jax.numpy as jnp
from jax import lax
from jax.experimental import pallas as pl
from jax.experimental.pallas import tpu as pltpu
from jax.experimental.pallas import tpu_sc as plsc

N = 50000
C = 128
H = 4
D = 32
E = 250000

NCH = 12
CHS = 4176
WIN = 128
NWIN = 6
PT = WIN * NWIN
NTILE = 32
CAP = PT * NTILE
DR = 264
ACC_ROWS = 16 * DR
PADROWS = ACC_ROWS - CHS
ZROWS = 24
DENR = 136

_NT = ("user", "item")
_ETS = (("user", "to", "item"), ("item", "to", "user"))


_BM = 2000


def _proj_body(x_ref, w_ref, b_ref, q_ref, k_ref, v_ref):
    y = jnp.dot(x_ref[...], w_ref[...], preferred_element_type=jnp.float32)
    y = y + b_ref[...]
    q_ref[...] = y[:, 0:C]
    k_ref[...] = y[:, C:2 * C]
    v_ref[...] = y[:, 2 * C:3 * C]


def _proj(x, w, b):
    out = jax.ShapeDtypeStruct((N, C), jnp.float32)
    return pl.pallas_call(
        _proj_body,
        grid=(N // _BM,),
        in_specs=[
            pl.BlockSpec((_BM, C), lambda i: (i, 0)),
            pl.BlockSpec((C, 3 * C), lambda i: (0, 0)),
            pl.BlockSpec((1, 3 * C), lambda i: (0, 0)),
        ],
        out_specs=[pl.BlockSpec((_BM, C), lambda i: (i, 0))] * 3,
        out_shape=[out, out, out],
    )(x, w, b)


def _mm_body(x_ref, w_ref, b_ref, o_ref):
    o_ref[...] = (
        jnp.dot(x_ref[...], w_ref[...], preferred_element_type=jnp.float32)
        + b_ref[...]
    )


def _mm(x, w, b):
    return pl.pallas_call(
        _mm_body,
        grid=(N // _BM,),
        in_specs=[
            pl.BlockSpec((_BM, C), lambda i: (i, 0)),
            pl.BlockSpec((C, C), lambda i: (0, 0)),
            pl.BlockSpec((1, C), lambda i: (0, 0)),
        ],
        out_specs=pl.BlockSpec((_BM, C), lambda i: (i, 0)),
        out_shape=jax.ShapeDtypeStruct((N, C), jnp.float32),
    )(x, w, b)


def _denmerge_body(d_ref, o_ref):
    s = jnp.zeros((1, 1, DENR, C), jnp.float32)
    for b in range(16):
        s = s + d_ref[:, b]
    o_ref[...] = s


def _denmerge(dout):
    return pl.pallas_call(
        _denmerge_body,
        grid=(2, NCH),
        in_specs=[pl.BlockSpec((1, 16, 1, DENR, C),
                               lambda t, c: (t, 0, c, 0, 0))],
        out_specs=pl.BlockSpec((1, 1, DENR, C), lambda t, c: (t, c, 0, 0)),
        out_shape=jax.ShapeDtypeStruct((2, NCH, DENR, C), jnp.float32),
    )(dout)


def _post_body(relu, n0, d4, xp, ex8, wa, ba, beta, o_ref):
    denb = jnp.dot(d4[...], ex8[0:H, :],
                   preferred_element_type=jnp.float32) + 1e-16
    agg = n0[...] / denb
    g = jax.nn.gelu(agg)
    out = (jnp.dot(g, wa[...], preferred_element_type=jnp.float32)
           + ba[...] + xp[...] * beta[...])
    if relu:
        out = jnp.maximum(out, 0.0)
    o_ref[...] = out


def _post(num, den4, x_prev, ex8, wa_eff, ba_eff, beta, relu):
    body = functools.partial(_post_body, relu)
    row = pl.BlockSpec((_BM, C), lambda i: (i, 0))
    den = pl.BlockSpec((_BM, H), lambda i: (i, 0))
    full = lambda r, c: pl.BlockSpec((r, c), lambda i: (0, 0))
    return pl.pallas_call(
        body,
        grid=(N // _BM,),
        in_specs=[row, den, row, full(8, C), full(C, C),
                  full(1, C), full(1, C)],
        out_specs=row,
        out_shape=jax.ShapeDtypeStruct((N, C), jnp.float32),
    )(num, den4, x_prev, ex8, wa_eff, ba_eff, beta)


PT2 = CAP // 16
NWIN2 = PT2 // WIN


def _sc_edge_body(q0, k0, v0, ss0, dg0, dl0, q1, k1, v1, ss1, dg1, dl1,
                  out, dout, acc, zb, qb, kb, vb, dt, ssb, dgb, dlb,
                  s1, s2, s3):
    ci = lax.axis_index("c")
    si = lax.axis_index("s")
    io = lax.broadcasted_iota(jnp.int32, (16,), 0)
    zeros16 = jnp.zeros((16,), jnp.float32)

    def zb_row(r, carry):
        for j in range(8):
            zb[r, pl.ds(j * 16, 16)] = zeros16
        return carry

    lax.fori_loop(0, ZROWS, zb_row, 0)

    def dt_row(r, carry):
        for j in range(8):
            dt[r, pl.ds(j * 16, 16)] = zeros16
        return carry

    def edge_body(e):
        exv = []
        for h in range(H):
            u = (qb[e, pl.ds(h * 32, 16)] * kb[e, pl.ds(h * 32, 16)]
                 + qb[e, pl.ds(h * 32 + 16, 16)] * kb[e, pl.ds(h * 32 + 16, 16)])
            s = jnp.sum(u)
            exv.append(jnp.exp(jnp.broadcast_to(s, (16,))))
        for j in range(2 * H):
            vb[e, pl.ds(j * 16, 16)] = vb[e, pl.ds(j * 16, 16)] * exv[j // 2]
        exl = jnp.where(io == 0, exv[0],
                        jnp.where(io == 1, exv[1],
                                  jnp.where(io == 2, exv[2], exv[3])))
        ev = jnp.broadcast_to(e, (16,)).astype(jnp.int32)
        dlv = plsc.load_gather(dlb, [ev])
        f = dlv * 4 + io
        plsc.addupdate_scatter(
            dt, [lax.shift_right_logical(f, 7), lax.bitwise_and(f, 127)],
            exl, mask=io < H)

    def run(q, kr, vr, ss, dg, dl):
        def win_body(c, w, carry):
            start = pl.multiple_of(c * CAP + si * PT2 + w * WIN, WIN)
            pltpu.sync_copy(ss.at[pl.ds(start, WIN)], ssb)
            pltpu.sync_copy(dg.at[pl.ds(start, WIN)], dgb)
            pltpu.sync_copy(dl.at[pl.ds(start, WIN)], dlb)
            h1 = pltpu.async_copy(q.at[dgb], qb, s1)
            h2 = pltpu.async_copy(kr.at[ssb], kb, s2)
            h3 = pltpu.async_copy(vr.at[ssb], vb, s3)
            h1.wait()
            h2.wait()
            h3.wait()
            plsc.parallel_loop(0, WIN, unroll=2)(edge_body)
            pltpu.sync_copy(vb, acc.at[dlb], add=True)
            return carry

        def chunk_body(c, carry):
            for z in range(11):
                off = pl.multiple_of(si * DR + z * ZROWS, 8)
                pltpu.sync_copy(zb, acc.at[pl.ds(off, ZROWS), :])
            lax.fori_loop(0, DENR, dt_row, 0)
            plsc.subcore_barrier()
            lax.fori_loop(0, NWIN2, functools.partial(win_body, c), 0)
            pltpu.sync_copy(dt, dout.at[ci, si, c])
            plsc.subcore_barrier()
            doff = pl.multiple_of(si * DR, 8)
            pltpu.sync_copy(acc.at[pl.ds(doff, DR), :],
                            out.at[ci, c, pl.ds(doff, DR), :])
            plsc.subcore_barrier()
            return carry

        lax.fori_loop(0, NCH, chunk_body, 0)

    @pl.when(ci == 0)
    def _():
        run(q0, k0, v0, ss0, dg0, dl0)

    @pl.when(ci == 1)
    def _():
        run(q1, k1, v1, ss1, dg1, dl1)


@functools.cache
def _sc_edge_kernel():
    return pl.kernel(
        _sc_edge_body,
        mesh=plsc.VectorSubcoreMesh(core_axis_name="c", subcore_axis_name="s",
                                    num_cores=2, num_subcores=16),
        out_type=[
            jax.ShapeDtypeStruct((2, NCH, ACC_ROWS, C), jnp.float32),
            jax.ShapeDtypeStruct((2, 16, NCH, DENR, C), jnp.float32),
        ],
        scratch_types=[
            pltpu.VMEM_SHARED((ACC_ROWS, C), jnp.float32),
            pltpu.VMEM((ZROWS, C), jnp.float32),
            pltpu.VMEM((WIN, C), jnp.float32),
            pltpu.VMEM((WIN, C), jnp.float32),
            pltpu.VMEM((WIN, C), jnp.float32),
            pltpu.VMEM((DENR, C), jnp.float32),
            pltpu.VMEM((WIN,), jnp.int32),
            pltpu.VMEM((WIN,), jnp.int32),
            pltpu.VMEM((WIN,), jnp.int32),
            pltpu.SemaphoreType.DMA,
            pltpu.SemaphoreType.DMA,
            pltpu.SemaphoreType.DMA,
        ],
        compiler_params=pltpu.CompilerParams(needs_layout_passes=False),
    )


def _sc_edge(args0, args1):
    return _sc_edge_kernel()(*args0, *args1)


def _partition_edges(ei):
    s, d = ei[0], ei[1]
    ch = (d // CHS).astype(jnp.int32)
    oh = (ch[:, None] == jnp.arange(NCH, dtype=jnp.int32)[None, :])
    oh = oh.astype(jnp.int32)
    pos_all = jnp.cumsum(oh, axis=0)
    pos = jnp.sum(pos_all * oh, axis=1) - 1
    tgt = jnp.where(pos < CAP, ch * CAP + pos, NCH * CAP)
    base = jnp.arange(NCH * CAP + 8, dtype=jnp.int32)
    ss = (base % N).at[tgt].add(s - tgt % N, unique_indices=True)
    dg = ((base * 17) % N).at[tgt].add(d - (tgt * 17) % N,
                                       unique_indices=True)
    dl = (CHS + base % PADROWS).at[tgt].add(
        d - ch * CHS - CHS - tgt % PADROWS, unique_indices=True)
    return ss[: NCH * CAP], dg[: NCH * CAP], dl[: NCH * CAP]


def _blockdiag(a):
    i = jnp.arange(H * D)
    hi = i // D
    mask = (hi[:, None] == hi[None, :]).astype(a.dtype)
    return a[hi[:, None], i[:, None] % D, i[None, :] % D] * mask


def _fold(p):
    f = {}
    for t in _NT:
        (et,) = [e for e in _ETS if e[0] == t]
        kk = "__".join(et)
        scale = (p["p_rel"][kk] / jnp.sqrt(float(D)))[:, None, None]
        ka = _blockdiag(p["a_rel"][kk] * scale)
        ma = _blockdiag(p["m_rel"][kk])
        wcat = jnp.concatenate(
            [p["W_q"][t], p["W_k"][t] @ ka, p["W_v"][t] @ ma], axis=1)
        bcat = jnp.concatenate(
            [p["b_q"][t], p["b_k"][t] @ ka, p["b_v"][t] @ ma])[None, :]
        a = jax.nn.sigmoid(p["skip"][t])
        f[t] = dict(
            wcat=wcat,
            bcat=bcat,
            wa=a * p["W_a"][t],
            ba=(a * p["b_a"][t])[None, :],
            beta=jnp.full((1, C), 1.0 - a, jnp.float32),
        )
    return f


def _split_acc(acc_t, dm_t):
    num = acc_t[:, :CHS, :].reshape(NCH * CHS, C)[:N]
    den = dm_t.reshape(NCH, DENR * C)[:, : ACC_ROWS * H]
    den = den.reshape(NCH, ACC_ROWS, H)[:, :CHS]
    den = den.reshape(NCH * CHS, H)[:N]
    return num, den


def kernel(x_user, x_item, edge_index_user_item, edge_index_item_user, params):
    part = {
        "user__to__item": _partition_edges(edge_index_user_item),
        "item__to__user": _partition_edges(edge_index_item_user),
    }
    ex8 = jnp.zeros((8, C), jnp.float32)
    ex8 = ex8.at[jnp.arange(C) // D, jnp.arange(C)].set(1.0)

    x = {"user": x_user, "item": x_item}
    for li, layer in enumerate(("conv1", "conv2")):
        f = _fold(params[layer])
        proj = {t: _proj(x[t], f[t]["wcat"], f[t]["bcat"]) for t in _NT}
        acc, dout = _sc_edge(
            (proj["item"][0], proj["user"][1], proj["user"][2],
             *part["user__to__item"]),
            (proj["user"][0], proj["item"][1], proj["item"][2],
             *part["item__to__user"]),
        )
        dm = _denmerge(dout)
        newx = {}
        for t, et_idx in (("item", 0), ("user", 1)):
            num, den4 = _split_acc(acc[et_idx], dm[et_idx])
            newx[t] = _post(num, den4, x[t], ex8, f[t]["wa"],
                            f[t]["ba"], f[t]["beta"], relu=(li == 0))
        x = newx

    lin_w = jnp.zeros((C, C), jnp.float32).at[:, :3].set(params["lin_W"])
    lin_b = jnp.zeros((1, C), jnp.float32).at[0, :3].set(params["lin_b"])
    return _mm(x["user"], lin_w, lin_b)[:, :3]


# speedup vs baseline: 1.1190x; 1.1190x over previous
import functools

import jax
import jax.numpy as jnp
from jax import lax
from jax.experimental import pallas as pl
from jax.experimental.pallas import tpu as pltpu
from jax.experimental.pallas import tpu_sc as plsc

N = 50000
C = 128
H = 4
D = 32
E = 250000

NCH = 12
CHS = 4176
WIN = 128
NWIN = 6
PT = WIN * NWIN
NTILE = 32
CAP = PT * NTILE
DR = 264
ACC_ROWS = 16 * DR
PADROWS = ACC_ROWS - CHS
ZROWS = 24
DENR = 136

_NT = ("user", "item")
_ETS = (("user", "to", "item"), ("item", "to", "user"))


_BM = 2000


def _proj_body(x_ref, w_ref, b_ref, q_ref, k_ref, v_ref):
    y = jnp.dot(x_ref[...], w_ref[...], preferred_element_type=jnp.float32)
    y = y + b_ref[...]
    q_ref[...] = y[:, 0:C]
    k_ref[...] = y[:, C:2 * C]
    v_ref[...] = y[:, 2 * C:3 * C]


def _proj(x, w, b):
    out = jax.ShapeDtypeStruct((N, C), jnp.float32)
    return pl.pallas_call(
        _proj_body,
        grid=(N // _BM,),
        in_specs=[
            pl.BlockSpec((_BM, C), lambda i: (i, 0)),
            pl.BlockSpec((C, 3 * C), lambda i: (0, 0)),
            pl.BlockSpec((1, 3 * C), lambda i: (0, 0)),
        ],
        out_specs=[pl.BlockSpec((_BM, C), lambda i: (i, 0))] * 3,
        out_shape=[out, out, out],
    )(x, w, b)


def _mm_body(x_ref, w_ref, b_ref, o_ref):
    o_ref[...] = (
        jnp.dot(x_ref[...], w_ref[...], preferred_element_type=jnp.float32)
        + b_ref[...]
    )


def _mm(x, w, b):
    return pl.pallas_call(
        _mm_body,
        grid=(N // _BM,),
        in_specs=[
            pl.BlockSpec((_BM, C), lambda i: (i, 0)),
            pl.BlockSpec((C, C), lambda i: (0, 0)),
            pl.BlockSpec((1, C), lambda i: (0, 0)),
        ],
        out_specs=pl.BlockSpec((_BM, C), lambda i: (i, 0)),
        out_shape=jax.ShapeDtypeStruct((N, C), jnp.float32),
    )(x, w, b)


def _denmerge_body(d_ref, o_ref):
    s = jnp.zeros((1, DENR, C), jnp.float32)
    for a in range(2):
        for b in range(16):
            s = s + d_ref[a, b]
    o_ref[...] = s


def _denmerge(dout):
    return pl.pallas_call(
        _denmerge_body,
        grid=(NCH,),
        in_specs=[pl.BlockSpec((2, 16, 1, DENR, C),
                               lambda c: (0, 0, c, 0, 0))],
        out_specs=pl.BlockSpec((1, DENR, C), lambda c: (c, 0, 0)),
        out_shape=jax.ShapeDtypeStruct((NCH, DENR, C), jnp.float32),
    )(dout)


def _post_body(relu, n0, n1, d4, xp, ex8, wa, ba, beta, o_ref):
    denb = jnp.dot(d4[...], ex8[0:H, :],
                   preferred_element_type=jnp.float32) + 1e-16
    agg = (n0[...] + n1[...]) / denb
    g = jax.nn.gelu(agg)
    out = (jnp.dot(g, wa[...], preferred_element_type=jnp.float32)
           + ba[...] + xp[...] * beta[...])
    if relu:
        out = jnp.maximum(out, 0.0)
    o_ref[...] = out


def _post(num_sc, den4, x_prev, ex8, wa_eff, ba_eff, beta, relu):
    body = functools.partial(_post_body, relu)
    row = pl.BlockSpec((_BM, C), lambda i: (i, 0))
    den = pl.BlockSpec((_BM, H), lambda i: (i, 0))
    full = lambda r, c: pl.BlockSpec((r, c), lambda i: (0, 0))
    return pl.pallas_call(
        body,
        grid=(N // _BM,),
        in_specs=[row, row, den, row, full(8, C), full(C, C),
                  full(1, C), full(1, C)],
        out_specs=row,
        out_shape=jax.ShapeDtypeStruct((N, C), jnp.float32),
    )(num_sc[0], num_sc[1], den4, x_prev, ex8, wa_eff, ba_eff, beta)


def _sc_edge_body(q, kr, vr, ss, dg, dl,
                  out, dout, acc, zb, qb, kb, vb, dt, ssb, dgb, dlb,
                  s1, s2, s3):
    ci = lax.axis_index("c")
    si = lax.axis_index("s")
    wid = ci * 16 + si
    io = lax.broadcasted_iota(jnp.int32, (16,), 0)
    zeros16 = jnp.zeros((16,), jnp.float32)

    def zb_row(r, carry):
        for j in range(8):
            zb[r, pl.ds(j * 16, 16)] = zeros16
        return carry

    lax.fori_loop(0, ZROWS, zb_row, 0)

    def dt_row(r, carry):
        for j in range(8):
            dt[r, pl.ds(j * 16, 16)] = zeros16
        return carry

    def edge_body(e):
        exv = []
        for h in range(H):
            u = (qb[e, pl.ds(h * 32, 16)] * kb[e, pl.ds(h * 32, 16)]
                 + qb[e, pl.ds(h * 32 + 16, 16)] * kb[e, pl.ds(h * 32 + 16, 16)])
            s = jnp.sum(u)
            exv.append(jnp.exp(jnp.broadcast_to(s, (16,))))
        for j in range(2 * H):
            vb[e, pl.ds(j * 16, 16)] = vb[e, pl.ds(j * 16, 16)] * exv[j // 2]
        exl = jnp.where(io == 0, exv[0],
                        jnp.where(io == 1, exv[1],
                                  jnp.where(io == 2, exv[2], exv[3])))
        ev = jnp.broadcast_to(e, (16,)).astype(jnp.int32)
        dlv = plsc.load_gather(dlb, [ev])
        f = dlv * 4 + io
        plsc.addupdate_scatter(
            dt, [lax.shift_right_logical(f, 7), lax.bitwise_and(f, 127)],
            exl, mask=io < H)

    def win_body(c, w, carry):
        start = pl.multiple_of(c * CAP + wid * PT + w * WIN, WIN)
        pltpu.sync_copy(ss.at[pl.ds(start, WIN)], ssb)
        pltpu.sync_copy(dg.at[pl.ds(start, WIN)], dgb)
        pltpu.sync_copy(dl.at[pl.ds(start, WIN)], dlb)
        h1 = pltpu.async_copy(q.at[dgb], qb, s1)
        h2 = pltpu.async_copy(kr.at[ssb], kb, s2)
        h3 = pltpu.async_copy(vr.at[ssb], vb, s3)
        h1.wait()
        h2.wait()
        h3.wait()
        plsc.parallel_loop(0, WIN, unroll=2)(edge_body)
        pltpu.sync_copy(vb, acc.at[dlb], add=True)
        return carry

    def chunk_body(c, carry):
        for z in range(11):
            off = pl.multiple_of(si * DR + z * ZROWS, 8)
            pltpu.sync_copy(zb, acc.at[pl.ds(off, ZROWS), :])
        lax.fori_loop(0, DENR, dt_row, 0)
        plsc.subcore_barrier()
        lax.fori_loop(0, NWIN, functools.partial(win_body, c), 0)
        pltpu.sync_copy(dt, dout.at[ci, si, c])
        plsc.subcore_barrier()
        doff = pl.multiple_of(si * DR, 8)
        pltpu.sync_copy(acc.at[pl.ds(doff, DR), :],
                        out.at[ci, c, pl.ds(doff, DR), :])
        plsc.subcore_barrier()
        return carry

    lax.fori_loop(0, NCH, chunk_body, 0)


@functools.cache
def _sc_edge_kernel():
    return pl.kernel(
        _sc_edge_body,
        mesh=plsc.VectorSubcoreMesh(core_axis_name="c", subcore_axis_name="s",
                                    num_cores=2, num_subcores=16),
        out_type=[
            jax.ShapeDtypeStruct((2, NCH, ACC_ROWS, C), jnp.float32),
            jax.ShapeDtypeStruct((2, 16, NCH, DENR, C), jnp.float32),
        ],
        scratch_types=[
            pltpu.VMEM_SHARED((ACC_ROWS, C), jnp.float32),
            pltpu.VMEM((ZROWS, C), jnp.float32),
            pltpu.VMEM((WIN, C), jnp.float32),
            pltpu.VMEM((WIN, C), jnp.float32),
            pltpu.VMEM((WIN, C), jnp.float32),
            pltpu.VMEM((DENR, C), jnp.float32),
            pltpu.VMEM((WIN,), jnp.int32),
            pltpu.VMEM((WIN,), jnp.int32),
            pltpu.VMEM((WIN,), jnp.int32),
            pltpu.SemaphoreType.DMA,
            pltpu.SemaphoreType.DMA,
            pltpu.SemaphoreType.DMA,
        ],
        compiler_params=pltpu.CompilerParams(needs_layout_passes=False),
    )


def _sc_edge(q, kr, vr, ss, dg, dl):
    return _sc_edge_kernel()(q, kr, vr, ss, dg, dl)


def _partition_edges(ei):
    s, d = ei[0], ei[1]
    ch = (d // CHS).astype(jnp.int32)
    oh = (ch[:, None] == jnp.arange(NCH, dtype=jnp.int32)[None, :])
    oh = oh.astype(jnp.int32)
    pos_all = jnp.cumsum(oh, axis=0)
    pos = jnp.sum(pos_all * oh, axis=1) - 1
    tgt = jnp.where(pos < CAP, ch * CAP + pos, NCH * CAP)
    base = jnp.arange(NCH * CAP + 8, dtype=jnp.int32)
    ss = (base % N).at[tgt].add(s - tgt % N, unique_indices=True)
    dg = ((base * 17) % N).at[tgt].add(d - (tgt * 17) % N,
                                       unique_indices=True)
    dl = (CHS + base % PADROWS).at[tgt].add(
        d - ch * CHS - CHS - tgt % PADROWS, unique_indices=True)
    return ss[: NCH * CAP], dg[: NCH * CAP], dl[: NCH * CAP]


def _blockdiag(a):
    i = jnp.arange(H * D)
    hi = i // D
    mask = (hi[:, None] == hi[None, :]).astype(a.dtype)
    return a[hi[:, None], i[:, None] % D, i[None, :] % D] * mask


def _fold(p):
    f = {}
    for t in _NT:
        (et,) = [e for e in _ETS if e[0] == t]
        kk = "__".join(et)
        scale = (p["p_rel"][kk] / jnp.sqrt(float(D)))[:, None, None]
        ka = _blockdiag(p["a_rel"][kk] * scale)
        ma = _blockdiag(p["m_rel"][kk])
        wcat = jnp.concatenate(
            [p["W_q"][t], p["W_k"][t] @ ka, p["W_v"][t] @ ma], axis=1)
        bcat = jnp.concatenate(
            [p["b_q"][t], p["b_k"][t] @ ka, p["b_v"][t] @ ma])[None, :]
        a = jax.nn.sigmoid(p["skip"][t])
        f[t] = dict(
            wcat=wcat,
            bcat=bcat,
            wa=a * p["W_a"][t],
            ba=(a * p["b_a"][t])[None, :],
            beta=jnp.full((1, C), 1.0 - a, jnp.float32),
        )
    return f


def _split_acc(acc, dmerged):
    num = acc[:, :, :CHS, :].reshape(2, NCH * CHS, C)[:, :N]
    den = dmerged.reshape(NCH, DENR * C)[:, : ACC_ROWS * H]
    den = den.reshape(NCH, ACC_ROWS, H)[:, :CHS]
    den = den.reshape(NCH * CHS, H)[:N]
    return num, den


def kernel(x_user, x_item, edge_index_user_item, edge_index_item_user, params):
    part = {
        "user__to__item": _partition_edges(edge_index_user_item),
        "item__to__user": _partition_edges(edge_index_item_user),
    }
    ex8 = jnp.zeros((8, C), jnp.float32)
    ex8 = ex8.at[jnp.arange(C) // D, jnp.arange(C)].set(1.0)

    x = {"user": x_user, "item": x_item}
    for li, layer in enumerate(("conv1", "conv2")):
        f = _fold(params[layer])
        proj = {t: _proj(x[t], f[t]["wcat"], f[t]["bcat"]) for t in _NT}
        newx = {}
        for et in _ETS:
            src, _, dst = et
            kk = "__".join(et)
            ss, dg, dl = part[kk]
            acc, dout = _sc_edge(proj[dst][0], proj[src][1], proj[src][2],
                                 ss, dg, dl)
            num, den4 = _split_acc(acc, _denmerge(dout))
            newx[dst] = _post(num, den4, x[dst], ex8, f[dst]["wa"],
                              f[dst]["ba"], f[dst]["beta"], relu=(li == 0))
        x = newx

    lin_w = jnp.zeros((C, C), jnp.float32).at[:, :3].set(params["lin_W"])
    lin_b = jnp.zeros((1, C), jnp.float32).at[0, :3].set(params["lin_b"])
    return _mm(x["user"], lin_w, lin_b)[:, :3]


# speedup vs baseline: 1.1774x; 1.0522x over previous
import functools

import jax
import jax.numpy as jnp
from jax import lax
from jax.experimental import pallas as pl
from jax.experimental.pallas import tpu as pltpu
from jax.experimental.pallas import tpu_sc as plsc

N = 50000
C = 128
H = 4
D = 32
E = 250000

NCH = 12
CHS = 4176
WIN = 128
NWIN = 6
PT = WIN * NWIN
NTILE = 32
CAP = PT * NTILE
DR = 264
ACC_ROWS = 16 * DR
PADROWS = ACC_ROWS - CHS
ZROWS = 24
DENR = 136

_NT = ("user", "item")
_ETS = (("user", "to", "item"), ("item", "to", "user"))


_BM = 2000


def _proj_body(x_ref, w_ref, b_ref, q_ref, k_ref, v_ref):
    y = jnp.dot(x_ref[...], w_ref[...], preferred_element_type=jnp.float32)
    y = y + b_ref[...]
    q_ref[...] = y[:, 0:C]
    k_ref[...] = y[:, C:2 * C]
    v_ref[...] = y[:, 2 * C:3 * C]


def _proj(x, w, b):
    out = jax.ShapeDtypeStruct((N, C), jnp.float32)
    return pl.pallas_call(
        _proj_body,
        grid=(N // _BM,),
        in_specs=[
            pl.BlockSpec((_BM, C), lambda i: (i, 0)),
            pl.BlockSpec((C, 3 * C), lambda i: (0, 0)),
            pl.BlockSpec((1, 3 * C), lambda i: (0, 0)),
        ],
        out_specs=[pl.BlockSpec((_BM, C), lambda i: (i, 0))] * 3,
        out_shape=[out, out, out],
    )(x, w, b)


def _mm_body(x_ref, w_ref, b_ref, o_ref):
    o_ref[...] = (
        jnp.dot(x_ref[...], w_ref[...], preferred_element_type=jnp.float32)
        + b_ref[...]
    )


def _mm(x, w, b):
    return pl.pallas_call(
        _mm_body,
        grid=(N // _BM,),
        in_specs=[
            pl.BlockSpec((_BM, C), lambda i: (i, 0)),
            pl.BlockSpec((C, C), lambda i: (0, 0)),
            pl.BlockSpec((1, C), lambda i: (0, 0)),
        ],
        out_specs=pl.BlockSpec((_BM, C), lambda i: (i, 0)),
        out_shape=jax.ShapeDtypeStruct((N, C), jnp.float32),
    )(x, w, b)


def _denmerge_body(d_ref, o_ref):
    s = jnp.zeros((1, DENR, C), jnp.float32)
    for a in range(2):
        for b in range(16):
            s = s + d_ref[a, b]
    o_ref[...] = s


def _denmerge(dout):
    return pl.pallas_call(
        _denmerge_body,
        grid=(NCH,),
        in_specs=[pl.BlockSpec((2, 16, 1, DENR, C),
                               lambda c: (0, 0, c, 0, 0))],
        out_specs=pl.BlockSpec((1, DENR, C), lambda c: (c, 0, 0)),
        out_shape=jax.ShapeDtypeStruct((NCH, DENR, C), jnp.float32),
    )(dout)


def _post_body(relu, n0, n1, d4, xp, ex8, wa, ba, beta, o_ref):
    denb = jnp.dot(d4[...], ex8[0:H, :],
                   preferred_element_type=jnp.float32) + 1e-16
    agg = (n0[...] + n1[...]) / denb
    g = jax.nn.gelu(agg)
    out = (jnp.dot(g, wa[...], preferred_element_type=jnp.float32)
           + ba[...] + xp[...] * beta[...])
    if relu:
        out = jnp.maximum(out, 0.0)
    o_ref[...] = out


def _post(num_sc, den4, x_prev, ex8, wa_eff, ba_eff, beta, relu):
    body = functools.partial(_post_body, relu)
    row = pl.BlockSpec((_BM, C), lambda i: (i, 0))
    den = pl.BlockSpec((_BM, H), lambda i: (i, 0))
    full = lambda r, c: pl.BlockSpec((r, c), lambda i: (0, 0))
    return pl.pallas_call(
        body,
        grid=(N // _BM,),
        in_specs=[row, row, den, row, full(8, C), full(C, C),
                  full(1, C), full(1, C)],
        out_specs=row,
        out_shape=jax.ShapeDtypeStruct((N, C), jnp.float32),
    )(num_sc[0], num_sc[1], den4, x_prev, ex8, wa_eff, ba_eff, beta)


def _sc_edge_body(q, kr, vr, ss, dg, dl,
                  out, dout, acc, zb, qb, kb, vb, dt, ssb, dgb, dlb,
                  s1, s2, s3):
    ci = lax.axis_index("c")
    si = lax.axis_index("s")
    wid = ci * 16 + si
    io = lax.broadcasted_iota(jnp.int32, (16,), 0)
    zeros16 = jnp.zeros((16,), jnp.float32)

    def zb_row(r, carry):
        for j in range(8):
            zb[r, pl.ds(j * 16, 16)] = zeros16
        return carry

    lax.fori_loop(0, ZROWS, zb_row, 0)

    def dt_row(r, carry):
        for j in range(8):
            dt[r, pl.ds(j * 16, 16)] = zeros16
        return carry

    def edge_body(e):
        exv = []
        for h in range(H):
            u = (qb[e, pl.ds(h * 32, 16)] * kb[e, pl.ds(h * 32, 16)]
                 + qb[e, pl.ds(h * 32 + 16, 16)] * kb[e, pl.ds(h * 32 + 16, 16)])
            s = jnp.sum(u)
            exv.append(jnp.exp(jnp.broadcast_to(s, (16,))))
        for j in range(2 * H):
            vb[e, pl.ds(j * 16, 16)] = vb[e, pl.ds(j * 16, 16)] * exv[j // 2]
        exl = jnp.where(io == 0, exv[0],
                        jnp.where(io == 1, exv[1],
                                  jnp.where(io == 2, exv[2], exv[3])))
        ev = jnp.broadcast_to(e, (16,)).astype(jnp.int32)
        dlv = plsc.load_gather(dlb, [ev])
        f = dlv * 4 + io
        plsc.addupdate_scatter(
            dt, [lax.shift_right_logical(f, 7), lax.bitwise_and(f, 127)],
            exl, mask=io < H)

    def win_body(c, w, carry):
        start = pl.multiple_of(c * CAP + wid * PT + w * WIN, WIN)
        pltpu.sync_copy(dl.at[pl.ds(start, WIN)], dlb)
        woff = pl.multiple_of(w * WIN, WIN)
        h1 = pltpu.async_copy(q.at[dgb.at[pl.ds(woff, WIN)]], qb, s1)
        h2 = pltpu.async_copy(kr.at[ssb.at[pl.ds(woff, WIN)]], kb, s2)
        h3 = pltpu.async_copy(vr.at[ssb.at[pl.ds(woff, WIN)]], vb, s3)
        h1.wait()
        h2.wait()
        h3.wait()
        plsc.parallel_loop(0, WIN, unroll=2)(edge_body)
        pltpu.sync_copy(vb, acc.at[dlb], add=True)
        return carry

    def chunk_body(c, carry):
        for z in range(11):
            off = pl.multiple_of(si * DR + z * ZROWS, 8)
            pltpu.sync_copy(zb, acc.at[pl.ds(off, ZROWS), :])
        lax.fori_loop(0, DENR, dt_row, 0)
        cbase = pl.multiple_of(c * CAP + wid * PT, WIN)
        pltpu.sync_copy(ss.at[pl.ds(cbase, PT)], ssb)
        pltpu.sync_copy(dg.at[pl.ds(cbase, PT)], dgb)
        plsc.subcore_barrier()
        lax.fori_loop(0, NWIN, functools.partial(win_body, c), 0)
        pltpu.sync_copy(dt, dout.at[ci, si, c])
        plsc.subcore_barrier()
        doff = pl.multiple_of(si * DR, 8)
        pltpu.sync_copy(acc.at[pl.ds(doff, DR), :],
                        out.at[ci, c, pl.ds(doff, DR), :])
        plsc.subcore_barrier()
        return carry

    lax.fori_loop(0, NCH, chunk_body, 0)


@functools.cache
def _sc_edge_kernel():
    return pl.kernel(
        _sc_edge_body,
        mesh=plsc.VectorSubcoreMesh(core_axis_name="c", subcore_axis_name="s",
                                    num_cores=2, num_subcores=16),
        out_type=[
            jax.ShapeDtypeStruct((2, NCH, ACC_ROWS, C), jnp.float32),
            jax.ShapeDtypeStruct((2, 16, NCH, DENR, C), jnp.float32),
        ],
        scratch_types=[
            pltpu.VMEM_SHARED((ACC_ROWS, C), jnp.float32),
            pltpu.VMEM((ZROWS, C), jnp.float32),
            pltpu.VMEM((WIN, C), jnp.float32),
            pltpu.VMEM((WIN, C), jnp.float32),
            pltpu.VMEM((WIN, C), jnp.float32),
            pltpu.VMEM((DENR, C), jnp.float32),
            pltpu.VMEM((PT,), jnp.int32),
            pltpu.VMEM((PT,), jnp.int32),
            pltpu.VMEM((WIN,), jnp.int32),
            pltpu.SemaphoreType.DMA,
            pltpu.SemaphoreType.DMA,
            pltpu.SemaphoreType.DMA,
        ],
        compiler_params=pltpu.CompilerParams(needs_layout_passes=False),
    )


def _sc_edge(q, kr, vr, ss, dg, dl):
    return _sc_edge_kernel()(q, kr, vr, ss, dg, dl)


def _partition_edges(ei):
    s, d = ei[0], ei[1]
    ch = (d // CHS).astype(jnp.int32)
    oh = (ch[:, None] == jnp.arange(NCH, dtype=jnp.int32)[None, :])
    oh = oh.astype(jnp.int32)
    pos_all = jnp.cumsum(oh, axis=0)
    pos = jnp.sum(pos_all * oh, axis=1) - 1
    tgt = jnp.where(pos < CAP, ch * CAP + pos, NCH * CAP)
    base = jnp.arange(NCH * CAP + 8, dtype=jnp.int32)
    ss = (base % N).at[tgt].add(s - tgt % N, unique_indices=True)
    dg = ((base * 17) % N).at[tgt].add(d - (tgt * 17) % N,
                                       unique_indices=True)
    dl = (CHS + base % PADROWS).at[tgt].add(
        d - ch * CHS - CHS - tgt % PADROWS, unique_indices=True)
    return ss[: NCH * CAP], dg[: NCH * CAP], dl[: NCH * CAP]


def _blockdiag(a):
    i = jnp.arange(H * D)
    hi = i // D
    mask = (hi[:, None] == hi[None, :]).astype(a.dtype)
    return a[hi[:, None], i[:, None] % D, i[None, :] % D] * mask


def _fold(p):
    f = {}
    for t in _NT:
        (et,) = [e for e in _ETS if e[0] == t]
        kk = "__".join(et)
        scale = (p["p_rel"][kk] / jnp.sqrt(float(D)))[:, None, None]
        ka = _blockdiag(p["a_rel"][kk] * scale)
        ma = _blockdiag(p["m_rel"][kk])
        wcat = jnp.concatenate(
            [p["W_q"][t], p["W_k"][t] @ ka, p["W_v"][t] @ ma], axis=1)
        bcat = jnp.concatenate(
            [p["b_q"][t], p["b_k"][t] @ ka, p["b_v"][t] @ ma])[None, :]
        a = jax.nn.sigmoid(p["skip"][t])
        f[t] = dict(
            wcat=wcat,
            bcat=bcat,
            wa=a * p["W_a"][t],
            ba=(a * p["b_a"][t])[None, :],
            beta=jnp.full((1, C), 1.0 - a, jnp.float32),
        )
    return f


def _split_acc(acc, dmerged):
    num = acc[:, :, :CHS, :].reshape(2, NCH * CHS, C)[:, :N]
    den = dmerged.reshape(NCH, DENR * C)[:, : ACC_ROWS * H]
    den = den.reshape(NCH, ACC_ROWS, H)[:, :CHS]
    den = den.reshape(NCH * CHS, H)[:N]
    return num, den


def kernel(x_user, x_item, edge_index_user_item, edge_index_item_user, params):
    part = {
        "user__to__item": _partition_edges(edge_index_user_item),
        "item__to__user": _partition_edges(edge_index_item_user),
    }
    ex8 = jnp.zeros((8, C), jnp.float32)
    ex8 = ex8.at[jnp.arange(C) // D, jnp.arange(C)].set(1.0)

    x = {"user": x_user, "item": x_item}
    for li, layer in enumerate(("conv1", "conv2")):
        f = _fold(params[layer])
        proj = {t: _proj(x[t], f[t]["wcat"], f[t]["bcat"]) for t in _NT}
        newx = {}
        for et in _ETS:
            src, _, dst = et
            kk = "__".join(et)
            ss, dg, dl = part[kk]
            acc, dout = _sc_edge(proj[dst][0], proj[src][1], proj[src][2],
                                 ss, dg, dl)
            num, den4 = _split_acc(acc, _denmerge(dout))
            newx[dst] = _post(num, den4, x[dst], ex8, f[dst]["wa"],
                              f[dst]["ba"], f[dst]["beta"], relu=(li == 0))
        x = newx

    lin_w = jnp.zeros((C, C), jnp.float32).at[:, :3].set(params["lin_W"])
    lin_b = jnp.zeros((1, C), jnp.float32).at[0, :3].set(params["lin_b"])
    return _mm(x["user"], lin_w, lin_b)[:, :3]
